# baseline (speedup 1.0000x reference)
import functools

import jax
import jax.numpy as jnp
from jax import lax
from jax.experimental import pallas as pl
from jax.experimental.pallas import tpu as pltpu
from jax.experimental.pallas import tpu_sc as plsc

NC = 2
NS = 16
LANES = 16
NW = NC * NS

F32 = jnp.float32


def _eproj_body(ea_ref, eh_ref, wep_ref, wgl_ref, bep_ref, bgl_ref, out_ref):
    wgl = wgl_ref[...]
    wc = jnp.dot(wep_ref[...], wgl, preferred_element_type=F32)
    b = jnp.dot(bep_ref[...], wgl, preferred_element_type=F32) + bgl_ref[...]
    out_ref[...] = (jnp.dot(ea_ref[...], wc, preferred_element_type=F32)
                    + jnp.dot(eh_ref[...], wgl, preferred_element_type=F32)
                    + b)


def _node_body(x_ref, a0_ref, a1_ref, a2_ref, a3_ref, w1_ref, b1_ref,
               w2_ref, b2_ref, gn_ref, bn_ref, out_ref):
    x = x_ref[...]
    h = x + (a0_ref[...] + a1_ref[...]) + (a2_ref[...] + a3_ref[...])
    h = jnp.maximum(jnp.dot(h, w1_ref[...], preferred_element_type=F32)
                    + b1_ref[...], 0.0)
    h = jnp.dot(h, w2_ref[...], preferred_element_type=F32) + b2_ref[...]
    mu = jnp.mean(h, axis=-1, keepdims=True)
    var = jnp.mean((h - mu) ** 2, axis=-1, keepdims=True)
    h = (h - mu) / jnp.sqrt(var + 1e-5) * gn_ref[...] + bn_ref[...]
    out_ref[...] = h + x


def _edge_body(hs_ref, hd_ref, ea_ref, eh_ref, wep_ref, bep_ref,
               wu1_ref, bu1_ref, wu2_ref, bu2_ref, ge_ref, be_ref, out_ref):
    hs = hs_ref[...]
    hd = hd_ref[...]
    e = (jnp.dot(ea_ref[...], wep_ref[...], preferred_element_type=F32)
         + bep_ref[...] + eh_ref[...])
    h = e.shape[-1]
    t = (jnp.dot(hs + hd, wu1_ref[0:h, :], preferred_element_type=F32)
         + jnp.dot(jnp.abs(hs - hd), wu1_ref[h:2 * h, :],
                   preferred_element_type=F32)
         + jnp.dot(e, wu1_ref[2 * h:3 * h, :], preferred_element_type=F32)
         + bu1_ref[...])
    t = jnp.maximum(t, 0.0)
    u = jnp.dot(t, wu2_ref[...], preferred_element_type=F32) + bu2_ref[...]
    mu = jnp.mean(u, axis=-1, keepdims=True)
    var = jnp.mean((u - mu) ** 2, axis=-1, keepdims=True)
    u = (u - mu) / jnp.sqrt(var + 1e-5) * ge_ref[...] + be_ref[...]
    out_ref[...] = u + e


def _make_scatter_kernel(e_half, n, h, soff):
    mb, sb = 400, 80
    msub = mb // sb
    per_w = e_half // NW
    nch = per_w // mb
    rps = ((n + NS * 8 - 1) // (NS * 8)) * 8
    np_ = rps * NS
    vpr = h // LANES
    mesh = plsc.VectorSubcoreMesh(core_axis_name="c", subcore_axis_name="s")

    @functools.partial(
        pl.kernel,
        out_type=jax.ShapeDtypeStruct((NC * np_, h), F32),
        mesh=mesh,
        scratch_types=[
            pltpu.VMEM((mb,), jnp.int32),
            pltpu.VMEM((msub, sb), jnp.int32),
            pltpu.VMEM((2, sb, h), F32),
            pltpu.VMEM((2, sb, h), F32),
            pltpu.VMEM_SHARED((np_, h), F32),
            [pltpu.SemaphoreType.DMA for _ in range(6)],
        ],
    )
    def scatter_k(ep_hbm, x_hbm, src_hbm, dst_hbm, out_hbm,
                  src_v, dst_v, xs_v, mt_v, acc, sems):
        cid = lax.axis_index("c")
        sid = lax.axis_index("s")
        wid = sid * NC + cid
        gsem = sems[0:2]
        esem = sems[2:4]
        ssem = sems[4:6]

        def zbody(r, _):
            for k in range(vpr):
                xs_v[0, r, pl.ds(k * LANES, LANES)] = jnp.zeros((LANES,), F32)
            return 0
        lax.fori_loop(0, sb, zbody, 0)
        done = 0
        while done < rps:
            pltpu.sync_copy(xs_v.at[0],
                            acc.at[pl.ds(sid * rps + done, sb)])
            done += sb
        plsc.subcore_barrier()

        def fire(base, j):
            p = j % 2
            g = pltpu.async_copy(
                x_hbm.at[src_v.at[pl.ds(j * sb, sb)]], xs_v.at[p], gsem[p])
            ee = pltpu.async_copy(
                ep_hbm.at[pl.ds(base + j * sb, sb)], mt_v.at[p], esem[p])
            return g, ee

        def drain_scatter(p):
            pltpu.make_async_copy(ep_hbm.at[pl.ds(0, sb)], mt_v.at[p],
                                  ssem[p]).wait()

        def chunk(ci, _):
            base = pl.multiple_of(wid * per_w + ci * mb, 8)
            gbase = pl.multiple_of(soff + base, 8)

            @pl.when(ci > 0)
            def _():
                drain_scatter(0)
                drain_scatter(1)

            pltpu.sync_copy(src_hbm.at[pl.ds(gbase, mb)], src_v)
            pltpu.sync_copy(dst_hbm.at[wid, ci], dst_v)
            cps = [None] * (msub + 1)
            cps[0] = fire(base, 0)
            for j in range(msub):
                p = j % 2
                if j + 1 < msub:
                    if j >= 1:
                        drain_scatter((j + 1) % 2)
                    cps[j + 1] = fire(base, j + 1)
                g, ee = cps[j]
                g.wait()
                ee.wait()

                def rbody(r, _):
                    for rr in range(2):
                        for k in range(vpr):
                            c = k * LANES
                            row = 2 * r + rr
                            v = xs_v[p, row, pl.ds(c, LANES)] \
                                + mt_v[p, row, pl.ds(c, LANES)]
                            mt_v[p, row, pl.ds(c, LANES)] = \
                                jnp.maximum(v, 0.0)
                    return 0
                lax.fori_loop(0, sb // 2, rbody, 0)
                pltpu.async_copy(mt_v.at[p], acc.at[dst_v.at[j]],
                                 ssem[p], add=True)
            return 0
        lax.fori_loop(0, nch, chunk, 0)
        drain_scatter(0)
        drain_scatter(1)
        plsc.subcore_barrier()
        pltpu.sync_copy(acc.at[pl.ds(sid * rps, rps)],
                        out_hbm.at[pl.ds(cid * np_ + sid * rps, rps)])

    def call(ep, x, src, dst):
        dst_h = lax.slice(dst, (soff,), (soff + e_half,))
        return scatter_k(ep, x, src, dst_h.reshape(NW, nch, msub, sb))

    return call, np_


def _make_gather_kernel(e_half, n, h, soff):
    mb, sb = 400, 80
    msub = mb // sb
    per_w = e_half // NW
    nch = per_w // mb
    mesh = plsc.VectorSubcoreMesh(core_axis_name="c", subcore_axis_name="s")

    @functools.partial(
        pl.kernel,
        out_type=(jax.ShapeDtypeStruct((e_half, h), F32),
                  jax.ShapeDtypeStruct((e_half, h), F32)),
        mesh=mesh,
        scratch_types=[
            pltpu.VMEM((mb,), jnp.int32),
            pltpu.VMEM((mb,), jnp.int32),
            pltpu.VMEM((2, sb, h), F32),
            pltpu.VMEM((2, sb, h), F32),
            [pltpu.SemaphoreType.DMA for _ in range(4)],
        ],
    )
    def gather_k(xo_hbm, src_hbm, dst_hbm, hs_hbm, hd_hbm,
                 si_v, di_v, bs_v, bd_v, sems):
        wid = lax.axis_index("s") * NC + lax.axis_index("c")
        gsem = sems[0:2]
        wsem = sems[2:4]

        def fire(j):
            p = j % 2
            g1 = pltpu.async_copy(xo_hbm.at[si_v.at[pl.ds(j * sb, sb)]],
                                  bs_v.at[p], gsem[p])
            g2 = pltpu.async_copy(xo_hbm.at[di_v.at[pl.ds(j * sb, sb)]],
                                  bd_v.at[p], gsem[p])
            return g1, g2

        def drain_wb(p):
            pltpu.make_async_copy(bs_v.at[p], hs_hbm.at[pl.ds(0, sb)],
                                  wsem[p]).wait()
            pltpu.make_async_copy(bd_v.at[p], hd_hbm.at[pl.ds(0, sb)],
                                  wsem[p]).wait()

        def chunk(ci, _):
            base = pl.multiple_of(wid * per_w + ci * mb, 8)
            gbase = pl.multiple_of(soff + base, 8)

            @pl.when(ci > 0)
            def _():
                drain_wb(0)
                drain_wb(1)

            pltpu.sync_copy(src_hbm.at[pl.ds(gbase, mb)], si_v)
            pltpu.sync_copy(dst_hbm.at[pl.ds(gbase, mb)], di_v)
            cps = [None] * (msub + 1)
            cps[0] = fire(0)
            for j in range(msub):
                p = j % 2
                if j + 1 < msub:
                    if j >= 1:
                        drain_wb((j + 1) % 2)
                    cps[j + 1] = fire(j + 1)
                g1, g2 = cps[j]
                g1.wait()
                g2.wait()
                pltpu.async_copy(bs_v.at[p],
                                 hs_hbm.at[pl.ds(base + j * sb, sb)], wsem[p])
                pltpu.async_copy(bd_v.at[p],
                                 hd_hbm.at[pl.ds(base + j * sb, sb)], wsem[p])
            return 0
        lax.fori_loop(0, nch, chunk, 0)
        drain_wb(0)
        drain_wb(1)

    return gather_k


def kernel(x, edge_index, edge_attr, edge_h, Wep, bep, Wgl, bgl,
           W1, b1, W2, b2, gn, bn, Wu1, bu1, Wu2, bu2, ge, be):
    n, h = x.shape
    e, ed = edge_attr.shape
    src = edge_index[0]
    dst = edge_index[1]
    (bep2, bgl2, b12, b22, gn2, bn2, bu12, bu22, ge2, be2) = [
        v.reshape(1, h) for v in (bep, bgl, b1, b2, gn, bn, bu1, bu2, ge, be)]

    unit = 12800
    e0 = (11 * (e // unit) // 25) * unit
    e1 = e - e0
    eb = 1600
    rep = lambda i: (i, 0)
    fix = lambda i: (0, 0)

    def eproj_part(soff, esz):
        blk0 = soff // eb
        off = lambda i, blk0=blk0: (i + blk0, 0)
        return pl.pallas_call(
            _eproj_body,
            grid=(esz // eb,),
            in_specs=[
                pl.BlockSpec((eb, ed), off),
                pl.BlockSpec((eb, h), off),
                pl.BlockSpec((ed, h), fix),
                pl.BlockSpec((h, h), fix),
                pl.BlockSpec((1, h), fix),
                pl.BlockSpec((1, h), fix),
            ],
            out_specs=pl.BlockSpec((eb, h), rep),
            out_shape=jax.ShapeDtypeStruct((esz, h), F32),
        )(edge_attr, edge_h, Wep, Wgl, bep2, bgl2)

    scatter0, np_ = _make_scatter_kernel(e0, n, h, 0)
    scatter1, _ = _make_scatter_kernel(e1, n, h, e0)
    ep0 = eproj_part(0, e0)
    ag0 = scatter0(ep0, x, src, dst)
    ep1 = eproj_part(e0, e1)
    ag1 = scatter1(ep1, x, src, dst)
    a0 = ag0[0:n]
    a1 = ag0[np_:np_ + n]
    a2 = ag1[0:n]
    a3 = ag1[np_:np_ + n]

    nb = 2000
    x_out = pl.pallas_call(
        _node_body,
        grid=(n // nb,),
        in_specs=[
            pl.BlockSpec((nb, h), rep),
            pl.BlockSpec((nb, h), rep),
            pl.BlockSpec((nb, h), rep),
            pl.BlockSpec((nb, h), rep),
            pl.BlockSpec((nb, h), rep),
            pl.BlockSpec((h, h), fix),
            pl.BlockSpec((1, h), fix),
            pl.BlockSpec((h, h), fix),
            pl.BlockSpec((1, h), fix),
            pl.BlockSpec((1, h), fix),
            pl.BlockSpec((1, h), fix),
        ],
        out_specs=pl.BlockSpec((nb, h), rep),
        out_shape=jax.ShapeDtypeStruct((n, h), F32),
    )(x, a0, a1, a2, a3, W1, b12, W2, b22, gn2, bn2)

    def edge_part(soff, esz, hs_h, hd_h, prev):
        blk0 = soff // eb
        off = lambda i, blk0=blk0: (i + blk0, 0)
        in_specs = [
            pl.BlockSpec((eb, h), rep),
            pl.BlockSpec((eb, h), rep),
            pl.BlockSpec((eb, ed), off),
            pl.BlockSpec((eb, h), off),
            pl.BlockSpec((ed, h), fix),
            pl.BlockSpec((1, h), fix),
            pl.BlockSpec((3 * h, h), fix),
            pl.BlockSpec((1, h), fix),
            pl.BlockSpec((h, h), fix),
            pl.BlockSpec((1, h), fix),
            pl.BlockSpec((1, h), fix),
            pl.BlockSpec((1, h), fix),
        ]
        args = [hs_h, hd_h, edge_attr, edge_h, Wep, bep2, Wu1, bu12,
                Wu2, bu22, ge2, be2]
        kwargs = {}
        body = _edge_body
        if prev is not None:
            body = lambda p_ref, *refs: _edge_body(*refs)
            in_specs = [pl.BlockSpec(memory_space=pl.ANY)] + in_specs
            args = [prev] + args
            kwargs = dict(input_output_aliases={0: 0})
        return pl.pallas_call(
            body,
            grid=(esz // eb,),
            in_specs=in_specs,
            out_specs=pl.BlockSpec((eb, h), off),
            out_shape=jax.ShapeDtypeStruct((e, h), F32),
            **kwargs,
        )(*args)

    gather0 = _make_gather_kernel(e0, n, h, 0)
    gather1 = _make_gather_kernel(e1, n, h, e0)
    hs0, hd0 = gather0(x_out, src, dst)
    en0 = edge_part(0, e0, hs0, hd0, None)
    hs1, hd1 = gather1(x_out, src, dst)
    e_new = edge_part(e0, e1, hs1, hd1, en0)

    return (x_out, e_new)


# speedup vs baseline: 1.0400x; 1.0400x over previous
import functools

import jax
import jax.numpy as jnp
from jax import lax
from jax.experimental import pallas as pl
from jax.experimental.pallas import tpu as pltpu
from jax.experimental.pallas import tpu_sc as plsc

NC = 2
NS = 16
LANES = 16
NW = NC * NS

F32 = jnp.float32


def _eproj_body(ea_ref, eh_ref, wep_ref, wgl_ref, bep_ref, bgl_ref, out_ref):
    wgl = wgl_ref[...]
    wc = jnp.dot(wep_ref[...], wgl, preferred_element_type=F32)
    b = jnp.dot(bep_ref[...], wgl, preferred_element_type=F32) + bgl_ref[...]
    out_ref[...] = (jnp.dot(ea_ref[...], wc, preferred_element_type=F32)
                    + jnp.dot(eh_ref[...], wgl, preferred_element_type=F32)
                    + b)


def _node_body(x_ref, a0_ref, a1_ref, a2_ref, a3_ref, w1_ref, b1_ref,
               w2_ref, b2_ref, gn_ref, bn_ref, out_ref):
    x = x_ref[...]
    h = x + (a0_ref[...] + a1_ref[...]) + (a2_ref[...] + a3_ref[...])
    h = jnp.maximum(jnp.dot(h, w1_ref[...], preferred_element_type=F32)
                    + b1_ref[...], 0.0)
    h = jnp.dot(h, w2_ref[...], preferred_element_type=F32) + b2_ref[...]
    mu = jnp.mean(h, axis=-1, keepdims=True)
    var = jnp.mean((h - mu) ** 2, axis=-1, keepdims=True)
    h = (h - mu) / jnp.sqrt(var + 1e-5) * gn_ref[...] + bn_ref[...]
    out_ref[...] = h + x


def _edge_body(hs_ref, hd_ref, ea_ref, eh_ref, wep_ref, bep_ref,
               wu1_ref, bu1_ref, wu2_ref, bu2_ref, ge_ref, be_ref, out_ref):
    hs = hs_ref[...]
    hd = hd_ref[...]
    e = (jnp.dot(ea_ref[...], wep_ref[...], preferred_element_type=F32)
         + bep_ref[...] + eh_ref[...])
    h = e.shape[-1]
    t = (jnp.dot(hs + hd, wu1_ref[0:h, :], preferred_element_type=F32)
         + jnp.dot(jnp.abs(hs - hd), wu1_ref[h:2 * h, :],
                   preferred_element_type=F32)
         + jnp.dot(e, wu1_ref[2 * h:3 * h, :], preferred_element_type=F32)
         + bu1_ref[...])
    t = jnp.maximum(t, 0.0)
    u = jnp.dot(t, wu2_ref[...], preferred_element_type=F32) + bu2_ref[...]
    mu = jnp.mean(u, axis=-1, keepdims=True)
    var = jnp.mean((u - mu) ** 2, axis=-1, keepdims=True)
    u = (u - mu) / jnp.sqrt(var + 1e-5) * ge_ref[...] + be_ref[...]
    out_ref[...] = u + e


def _make_scatter_kernel(e_half, n, h, soff):
    mb, sb = 400, 80
    msub = mb // sb
    per_w = e_half // NW
    nch = per_w // mb
    rps = ((n + NS * 8 - 1) // (NS * 8)) * 8
    np_ = rps * NS
    vpr = h // LANES
    mesh = plsc.VectorSubcoreMesh(core_axis_name="c", subcore_axis_name="s")

    @functools.partial(
        pl.kernel,
        out_type=jax.ShapeDtypeStruct((NC * np_, h), F32),
        mesh=mesh,
        scratch_types=[
            pltpu.VMEM((mb,), jnp.int32),
            pltpu.VMEM((msub, sb), jnp.int32),
            pltpu.VMEM((2, sb, h), F32),
            pltpu.VMEM((2, sb, h), F32),
            pltpu.VMEM_SHARED((np_, h), F32),
            [pltpu.SemaphoreType.DMA for _ in range(6)],
        ],
    )
    def scatter_k(ep_hbm, x_hbm, src_hbm, dst_hbm, out_hbm,
                  src_v, dst_v, xs_v, mt_v, acc, sems):
        cid = lax.axis_index("c")
        sid = lax.axis_index("s")
        wid = sid * NC + cid
        gsem = sems[0:2]
        esem = sems[2:4]
        ssem = sems[4:6]

        def zbody(r, _):
            for k in range(vpr):
                xs_v[0, r, pl.ds(k * LANES, LANES)] = jnp.zeros((LANES,), F32)
            return 0
        lax.fori_loop(0, sb, zbody, 0)
        done = 0
        while done < rps:
            pltpu.sync_copy(xs_v.at[0],
                            acc.at[pl.ds(sid * rps + done, sb)])
            done += sb
        plsc.subcore_barrier()

        def fire(base, j):
            p = j % 2
            g = pltpu.async_copy(
                x_hbm.at[src_v.at[pl.ds(j * sb, sb)]], xs_v.at[p], gsem[p])
            ee = pltpu.async_copy(
                ep_hbm.at[pl.ds(base + j * sb, sb)], mt_v.at[p], esem[p])
            return g, ee

        def drain_scatter(p):
            pltpu.make_async_copy(ep_hbm.at[pl.ds(0, sb)], mt_v.at[p],
                                  ssem[p]).wait()

        def chunk(ci, _):
            base = pl.multiple_of(wid * per_w + ci * mb, 8)
            gbase = pl.multiple_of(soff + base, 8)

            @pl.when(ci > 0)
            def _():
                drain_scatter(0)
                drain_scatter(1)

            pltpu.sync_copy(src_hbm.at[pl.ds(gbase, mb)], src_v)
            pltpu.sync_copy(dst_hbm.at[wid, ci], dst_v)
            cps = [None] * (msub + 1)
            cps[0] = fire(base, 0)
            for j in range(msub):
                p = j % 2
                if j + 1 < msub:
                    if j >= 1:
                        drain_scatter((j + 1) % 2)
                    cps[j + 1] = fire(base, j + 1)
                g, ee = cps[j]
                g.wait()
                ee.wait()

                def rbody(r, _):
                    for rr in range(2):
                        for k in range(vpr):
                            c = k * LANES
                            row = 2 * r + rr
                            v = xs_v[p, row, pl.ds(c, LANES)] \
                                + mt_v[p, row, pl.ds(c, LANES)]
                            mt_v[p, row, pl.ds(c, LANES)] = \
                                jnp.maximum(v, 0.0)
                    return 0
                lax.fori_loop(0, sb // 2, rbody, 0)
                pltpu.async_copy(mt_v.at[p], acc.at[dst_v.at[j]],
                                 ssem[p], add=True)
            return 0
        lax.fori_loop(0, nch, chunk, 0)
        drain_scatter(0)
        drain_scatter(1)
        plsc.subcore_barrier()
        pltpu.sync_copy(acc.at[pl.ds(sid * rps, rps)],
                        out_hbm.at[pl.ds(cid * np_ + sid * rps, rps)])

    def call(ep, x, src, dst):
        dst_h = lax.slice(dst, (soff,), (soff + e_half,))
        return scatter_k(ep, x, src, dst_h.reshape(NW, nch, msub, sb))

    return call, np_


def _make_gather_kernel(e_half, n, h, soff):
    mb, sb = 400, 80
    msub = mb // sb
    per_w = e_half // NW
    nch = per_w // mb
    mesh = plsc.VectorSubcoreMesh(core_axis_name="c", subcore_axis_name="s")

    @functools.partial(
        pl.kernel,
        out_type=(jax.ShapeDtypeStruct((e_half, h), F32),
                  jax.ShapeDtypeStruct((e_half, h), F32)),
        mesh=mesh,
        scratch_types=[
            pltpu.VMEM((mb,), jnp.int32),
            pltpu.VMEM((mb,), jnp.int32),
            pltpu.VMEM((2, sb, h), F32),
            pltpu.VMEM((2, sb, h), F32),
            [pltpu.SemaphoreType.DMA for _ in range(4)],
        ],
    )
    def gather_k(xo_hbm, src_hbm, dst_hbm, hs_hbm, hd_hbm,
                 si_v, di_v, bs_v, bd_v, sems):
        wid = lax.axis_index("s") * NC + lax.axis_index("c")
        gsem = sems[0:2]
        wsem = sems[2:4]

        def fire(j):
            p = j % 2
            g1 = pltpu.async_copy(xo_hbm.at[si_v.at[pl.ds(j * sb, sb)]],
                                  bs_v.at[p], gsem[p])
            g2 = pltpu.async_copy(xo_hbm.at[di_v.at[pl.ds(j * sb, sb)]],
                                  bd_v.at[p], gsem[p])
            return g1, g2

        def drain_wb(p):
            pltpu.make_async_copy(bs_v.at[p], hs_hbm.at[pl.ds(0, sb)],
                                  wsem[p]).wait()
            pltpu.make_async_copy(bd_v.at[p], hd_hbm.at[pl.ds(0, sb)],
                                  wsem[p]).wait()

        def chunk(ci, _):
            base = pl.multiple_of(wid * per_w + ci * mb, 8)
            gbase = pl.multiple_of(soff + base, 8)

            @pl.when(ci > 0)
            def _():
                drain_wb(0)
                drain_wb(1)

            pltpu.sync_copy(src_hbm.at[pl.ds(gbase, mb)], si_v)
            pltpu.sync_copy(dst_hbm.at[pl.ds(gbase, mb)], di_v)
            cps = [None] * (msub + 1)
            cps[0] = fire(0)
            for j in range(msub):
                p = j % 2
                if j + 1 < msub:
                    if j >= 1:
                        drain_wb((j + 1) % 2)
                    cps[j + 1] = fire(j + 1)
                g1, g2 = cps[j]
                g1.wait()
                g2.wait()
                pltpu.async_copy(bs_v.at[p],
                                 hs_hbm.at[pl.ds(base + j * sb, sb)], wsem[p])
                pltpu.async_copy(bd_v.at[p],
                                 hd_hbm.at[pl.ds(base + j * sb, sb)], wsem[p])
            return 0
        lax.fori_loop(0, nch, chunk, 0)
        drain_wb(0)
        drain_wb(1)

    return gather_k


def kernel(x, edge_index, edge_attr, edge_h, Wep, bep, Wgl, bgl,
           W1, b1, W2, b2, gn, bn, Wu1, bu1, Wu2, bu2, ge, be):
    n, h = x.shape
    e, ed = edge_attr.shape
    src = edge_index[0]
    dst = edge_index[1]
    (bep2, bgl2, b12, b22, gn2, bn2, bu12, bu22, ge2, be2) = [
        v.reshape(1, h) for v in (bep, bgl, b1, b2, gn, bn, bu1, bu2, ge, be)]

    unit = 16000
    e0 = (2 * (e // unit) // 5) * unit
    e1 = e - e0
    eb = 2000
    rep = lambda i: (i, 0)
    fix = lambda i: (0, 0)

    def eproj_part(soff, esz):
        blk0 = soff // eb
        off = lambda i, blk0=blk0: (i + blk0, 0)
        return pl.pallas_call(
            _eproj_body,
            grid=(esz // eb,),
            in_specs=[
                pl.BlockSpec((eb, ed), off),
                pl.BlockSpec((eb, h), off),
                pl.BlockSpec((ed, h), fix),
                pl.BlockSpec((h, h), fix),
                pl.BlockSpec((1, h), fix),
                pl.BlockSpec((1, h), fix),
            ],
            out_specs=pl.BlockSpec((eb, h), rep),
            out_shape=jax.ShapeDtypeStruct((esz, h), F32),
        )(edge_attr, edge_h, Wep, Wgl, bep2, bgl2)

    scatter0, np_ = _make_scatter_kernel(e0, n, h, 0)
    scatter1, _ = _make_scatter_kernel(e1, n, h, e0)
    ep0 = eproj_part(0, e0)
    ag0 = scatter0(ep0, x, src, dst)
    ep1 = eproj_part(e0, e1)
    ag1 = scatter1(ep1, x, src, dst)
    a0 = ag0[0:n]
    a1 = ag0[np_:np_ + n]
    a2 = ag1[0:n]
    a3 = ag1[np_:np_ + n]

    nb = 2000
    x_out = pl.pallas_call(
        _node_body,
        grid=(n // nb,),
        in_specs=[
            pl.BlockSpec((nb, h), rep),
            pl.BlockSpec((nb, h), rep),
            pl.BlockSpec((nb, h), rep),
            pl.BlockSpec((nb, h), rep),
            pl.BlockSpec((nb, h), rep),
            pl.BlockSpec((h, h), fix),
            pl.BlockSpec((1, h), fix),
            pl.BlockSpec((h, h), fix),
            pl.BlockSpec((1, h), fix),
            pl.BlockSpec((1, h), fix),
            pl.BlockSpec((1, h), fix),
        ],
        out_specs=pl.BlockSpec((nb, h), rep),
        out_shape=jax.ShapeDtypeStruct((n, h), F32),
    )(x, a0, a1, a2, a3, W1, b12, W2, b22, gn2, bn2)

    def edge_part(soff, esz, hs_h, hd_h, prev):
        blk0 = soff // eb
        off = lambda i, blk0=blk0: (i + blk0, 0)
        in_specs = [
            pl.BlockSpec((eb, h), rep),
            pl.BlockSpec((eb, h), rep),
            pl.BlockSpec((eb, ed), off),
            pl.BlockSpec((eb, h), off),
            pl.BlockSpec((ed, h), fix),
            pl.BlockSpec((1, h), fix),
            pl.BlockSpec((3 * h, h), fix),
            pl.BlockSpec((1, h), fix),
            pl.BlockSpec((h, h), fix),
            pl.BlockSpec((1, h), fix),
            pl.BlockSpec((1, h), fix),
            pl.BlockSpec((1, h), fix),
        ]
        args = [hs_h, hd_h, edge_attr, edge_h, Wep, bep2, Wu1, bu12,
                Wu2, bu22, ge2, be2]
        kwargs = {}
        body = _edge_body
        if prev is not None:
            body = lambda p_ref, *refs: _edge_body(*refs)
            in_specs = [pl.BlockSpec(memory_space=pl.ANY)] + in_specs
            args = [prev] + args
            kwargs = dict(input_output_aliases={0: 0})
        return pl.pallas_call(
            body,
            grid=(esz // eb,),
            in_specs=in_specs,
            out_specs=pl.BlockSpec((eb, h), off),
            out_shape=jax.ShapeDtypeStruct((e, h), F32),
            **kwargs,
        )(*args)

    gather0 = _make_gather_kernel(e0, n, h, 0)
    gather1 = _make_gather_kernel(e1, n, h, e0)
    hs0, hd0 = gather0(x_out, src, dst)
    en0 = edge_part(0, e0, hs0, hd0, None)
    hs1, hd1 = gather1(x_out, src, dst)
    e_new = edge_part(e0, e1, hs1, hd1, en0)

    return (x_out, e_new)


# speedup vs baseline: 1.0508x; 1.0104x over previous
import functools

import jax
import jax.numpy as jnp
from jax import lax
from jax.experimental import pallas as pl
from jax.experimental.pallas import tpu as pltpu
from jax.experimental.pallas import tpu_sc as plsc

NC = 2
NS = 16
LANES = 16
NW = NC * NS

F32 = jnp.float32


def _eproj_body(ea_ref, eh_ref, wep_ref, wgl_ref, bep_ref, bgl_ref, out_ref):
    wgl = wgl_ref[...]
    wc = jnp.dot(wep_ref[...], wgl, preferred_element_type=F32)
    b = jnp.dot(bep_ref[...], wgl, preferred_element_type=F32) + bgl_ref[...]
    out_ref[...] = (jnp.dot(ea_ref[...], wc, preferred_element_type=F32)
                    + jnp.dot(eh_ref[...], wgl, preferred_element_type=F32)
                    + b)


def _node_body(x_ref, a0_ref, a1_ref, a2_ref, a3_ref, a4_ref, a5_ref,
               w1_ref, b1_ref, w2_ref, b2_ref, gn_ref, bn_ref, out_ref):
    x = x_ref[...]
    h = (x + (a0_ref[...] + a1_ref[...]) + (a2_ref[...] + a3_ref[...])
         + (a4_ref[...] + a5_ref[...]))
    h = jnp.maximum(jnp.dot(h, w1_ref[...], preferred_element_type=F32)
                    + b1_ref[...], 0.0)
    h = jnp.dot(h, w2_ref[...], preferred_element_type=F32) + b2_ref[...]
    mu = jnp.mean(h, axis=-1, keepdims=True)
    var = jnp.mean((h - mu) ** 2, axis=-1, keepdims=True)
    h = (h - mu) / jnp.sqrt(var + 1e-5) * gn_ref[...] + bn_ref[...]
    out_ref[...] = h + x


def _edge_body(hs_ref, hd_ref, ea_ref, eh_ref, wep_ref, bep_ref,
               wu1_ref, bu1_ref, wu2_ref, bu2_ref, ge_ref, be_ref, out_ref):
    hs = hs_ref[...]
    hd = hd_ref[...]
    e = (jnp.dot(ea_ref[...], wep_ref[...], preferred_element_type=F32)
         + bep_ref[...] + eh_ref[...])
    h = e.shape[-1]
    t = (jnp.dot(hs + hd, wu1_ref[0:h, :], preferred_element_type=F32)
         + jnp.dot(jnp.abs(hs - hd), wu1_ref[h:2 * h, :],
                   preferred_element_type=F32)
         + jnp.dot(e, wu1_ref[2 * h:3 * h, :], preferred_element_type=F32)
         + bu1_ref[...])
    t = jnp.maximum(t, 0.0)
    u = jnp.dot(t, wu2_ref[...], preferred_element_type=F32) + bu2_ref[...]
    mu = jnp.mean(u, axis=-1, keepdims=True)
    var = jnp.mean((u - mu) ** 2, axis=-1, keepdims=True)
    u = (u - mu) / jnp.sqrt(var + 1e-5) * ge_ref[...] + be_ref[...]
    out_ref[...] = u + e


def _make_scatter_kernel(e_half, n, h, soff):
    mb, sb = 400, 80
    msub = mb // sb
    per_w = e_half // NW
    nch = per_w // mb
    rps = ((n + NS * 8 - 1) // (NS * 8)) * 8
    np_ = rps * NS
    vpr = h // LANES
    mesh = plsc.VectorSubcoreMesh(core_axis_name="c", subcore_axis_name="s")

    @functools.partial(
        pl.kernel,
        out_type=jax.ShapeDtypeStruct((NC * np_, h), F32),
        mesh=mesh,
        scratch_types=[
            pltpu.VMEM((mb,), jnp.int32),
            pltpu.VMEM((msub, sb), jnp.int32),
            pltpu.VMEM((2, sb, h), F32),
            pltpu.VMEM((2, sb, h), F32),
            pltpu.VMEM_SHARED((np_, h), F32),
            [pltpu.SemaphoreType.DMA for _ in range(6)],
        ],
    )
    def scatter_k(ep_hbm, x_hbm, src_hbm, dst_hbm, out_hbm,
                  src_v, dst_v, xs_v, mt_v, acc, sems):
        cid = lax.axis_index("c")
        sid = lax.axis_index("s")
        wid = sid * NC + cid
        gsem = sems[0:2]
        esem = sems[2:4]
        ssem = sems[4:6]

        def zbody(r, _):
            for k in range(vpr):
                xs_v[0, r, pl.ds(k * LANES, LANES)] = jnp.zeros((LANES,), F32)
            return 0
        lax.fori_loop(0, sb, zbody, 0)
        done = 0
        while done < rps:
            pltpu.sync_copy(xs_v.at[0],
                            acc.at[pl.ds(sid * rps + done, sb)])
            done += sb
        plsc.subcore_barrier()

        def fire(base, j):
            p = j % 2
            g = pltpu.async_copy(
                x_hbm.at[src_v.at[pl.ds(j * sb, sb)]], xs_v.at[p], gsem[p])
            ee = pltpu.async_copy(
                ep_hbm.at[pl.ds(base + j * sb, sb)], mt_v.at[p], esem[p])
            return g, ee

        def drain_scatter(p):
            pltpu.make_async_copy(ep_hbm.at[pl.ds(0, sb)], mt_v.at[p],
                                  ssem[p]).wait()

        def chunk(ci, _):
            base = pl.multiple_of(wid * per_w + ci * mb, 8)
            gbase = pl.multiple_of(soff + base, 8)

            @pl.when(ci > 0)
            def _():
                drain_scatter(0)
                drain_scatter(1)

            pltpu.sync_copy(src_hbm.at[pl.ds(gbase, mb)], src_v)
            pltpu.sync_copy(dst_hbm.at[wid, ci], dst_v)
            cps = [None] * (msub + 1)
            cps[0] = fire(base, 0)
            for j in range(msub):
                p = j % 2
                if j + 1 < msub:
                    if j >= 1:
                        drain_scatter((j + 1) % 2)
                    cps[j + 1] = fire(base, j + 1)
                g, ee = cps[j]
                g.wait()
                ee.wait()

                def rbody(r, _):
                    for rr in range(2):
                        for k in range(vpr):
                            c = k * LANES
                            row = 2 * r + rr
                            v = xs_v[p, row, pl.ds(c, LANES)] \
                                + mt_v[p, row, pl.ds(c, LANES)]
                            mt_v[p, row, pl.ds(c, LANES)] = \
                                jnp.maximum(v, 0.0)
                    return 0
                lax.fori_loop(0, sb // 2, rbody, 0)
                pltpu.async_copy(mt_v.at[p], acc.at[dst_v.at[j]],
                                 ssem[p], add=True)
            return 0
        lax.fori_loop(0, nch, chunk, 0)
        drain_scatter(0)
        drain_scatter(1)
        plsc.subcore_barrier()
        pltpu.sync_copy(acc.at[pl.ds(sid * rps, rps)],
                        out_hbm.at[pl.ds(cid * np_ + sid * rps, rps)])

    def call(ep, x, src, dst):
        dst_h = lax.slice(dst, (soff,), (soff + e_half,))
        return scatter_k(ep, x, src, dst_h.reshape(NW, nch, msub, sb))

    return call, np_


def _make_gather_kernel(e_half, n, h, soff):
    mb, sb = 400, 80
    msub = mb // sb
    per_w = e_half // NW
    nch = per_w // mb
    mesh = plsc.VectorSubcoreMesh(core_axis_name="c", subcore_axis_name="s")

    @functools.partial(
        pl.kernel,
        out_type=(jax.ShapeDtypeStruct((e_half, h), F32),
                  jax.ShapeDtypeStruct((e_half, h), F32)),
        mesh=mesh,
        scratch_types=[
            pltpu.VMEM((mb,), jnp.int32),
            pltpu.VMEM((mb,), jnp.int32),
            pltpu.VMEM((2, sb, h), F32),
            pltpu.VMEM((2, sb, h), F32),
            [pltpu.SemaphoreType.DMA for _ in range(4)],
        ],
    )
    def gather_k(xo_hbm, src_hbm, dst_hbm, hs_hbm, hd_hbm,
                 si_v, di_v, bs_v, bd_v, sems):
        wid = lax.axis_index("s") * NC + lax.axis_index("c")
        gsem = sems[0:2]
        wsem = sems[2:4]

        def fire(j):
            p = j % 2
            g1 = pltpu.async_copy(xo_hbm.at[si_v.at[pl.ds(j * sb, sb)]],
                                  bs_v.at[p], gsem[p])
            g2 = pltpu.async_copy(xo_hbm.at[di_v.at[pl.ds(j * sb, sb)]],
                                  bd_v.at[p], gsem[p])
            return g1, g2

        def drain_wb(p):
            pltpu.make_async_copy(bs_v.at[p], hs_hbm.at[pl.ds(0, sb)],
                                  wsem[p]).wait()
            pltpu.make_async_copy(bd_v.at[p], hd_hbm.at[pl.ds(0, sb)],
                                  wsem[p]).wait()

        def chunk(ci, _):
            base = pl.multiple_of(wid * per_w + ci * mb, 8)
            gbase = pl.multiple_of(soff + base, 8)

            @pl.when(ci > 0)
            def _():
                drain_wb(0)
                drain_wb(1)

            pltpu.sync_copy(src_hbm.at[pl.ds(gbase, mb)], si_v)
            pltpu.sync_copy(dst_hbm.at[pl.ds(gbase, mb)], di_v)
            cps = [None] * (msub + 1)
            cps[0] = fire(0)
            for j in range(msub):
                p = j % 2
                if j + 1 < msub:
                    if j >= 1:
                        drain_wb((j + 1) % 2)
                    cps[j + 1] = fire(j + 1)
                g1, g2 = cps[j]
                g1.wait()
                g2.wait()
                pltpu.async_copy(bs_v.at[p],
                                 hs_hbm.at[pl.ds(base + j * sb, sb)], wsem[p])
                pltpu.async_copy(bd_v.at[p],
                                 hd_hbm.at[pl.ds(base + j * sb, sb)], wsem[p])
            return 0
        lax.fori_loop(0, nch, chunk, 0)
        drain_wb(0)
        drain_wb(1)

    return gather_k


def kernel(x, edge_index, edge_attr, edge_h, Wep, bep, Wgl, bgl,
           W1, b1, W2, b2, gn, bn, Wu1, bu1, Wu2, bu2, ge, be):
    n, h = x.shape
    e, ed = edge_attr.shape
    src = edge_index[0]
    dst = edge_index[1]
    (bep2, bgl2, b12, b22, gn2, bn2, bu12, bu22, ge2, be2) = [
        v.reshape(1, h) for v in (bep, bgl, b1, b2, gn, bn, bu1, bu2, ge, be)]

    u5 = e // 5
    slices = [(0, u5), (u5, 2 * u5), (3 * u5, 2 * u5)]
    eb = 2000
    rep = lambda i: (i, 0)
    fix = lambda i: (0, 0)

    def eproj_part(soff, esz):
        blk0 = soff // eb
        off = lambda i, blk0=blk0: (i + blk0, 0)
        return pl.pallas_call(
            _eproj_body,
            grid=(esz // eb,),
            in_specs=[
                pl.BlockSpec((eb, ed), off),
                pl.BlockSpec((eb, h), off),
                pl.BlockSpec((ed, h), fix),
                pl.BlockSpec((h, h), fix),
                pl.BlockSpec((1, h), fix),
                pl.BlockSpec((1, h), fix),
            ],
            out_specs=pl.BlockSpec((eb, h), rep),
            out_shape=jax.ShapeDtypeStruct((esz, h), F32),
        )(edge_attr, edge_h, Wep, Wgl, bep2, bgl2)

    aggrs = []
    np_ = None
    for soff, esz in slices:
        scat, np_ = _make_scatter_kernel(esz, n, h, soff)
        aggrs.append(scat(eproj_part(soff, esz), x, src, dst))
    parts = [a for ag in aggrs for a in (ag[0:n], ag[np_:np_ + n])]

    nb = 2000
    x_out = pl.pallas_call(
        _node_body,
        grid=(n // nb,),
        in_specs=[pl.BlockSpec((nb, h), rep)] * 7 + [
            pl.BlockSpec((h, h), fix),
            pl.BlockSpec((1, h), fix),
            pl.BlockSpec((h, h), fix),
            pl.BlockSpec((1, h), fix),
            pl.BlockSpec((1, h), fix),
            pl.BlockSpec((1, h), fix),
        ],
        out_specs=pl.BlockSpec((nb, h), rep),
        out_shape=jax.ShapeDtypeStruct((n, h), F32),
    )(x, *parts, W1, b12, W2, b22, gn2, bn2)
    def edge_part(soff, esz, hs_h, hd_h, prev):
        blk0 = soff // eb
        off = lambda i, blk0=blk0: (i + blk0, 0)
        in_specs = [
            pl.BlockSpec((eb, h), rep),
            pl.BlockSpec((eb, h), rep),
            pl.BlockSpec((eb, ed), off),
            pl.BlockSpec((eb, h), off),
            pl.BlockSpec((ed, h), fix),
            pl.BlockSpec((1, h), fix),
            pl.BlockSpec((3 * h, h), fix),
            pl.BlockSpec((1, h), fix),
            pl.BlockSpec((h, h), fix),
            pl.BlockSpec((1, h), fix),
            pl.BlockSpec((1, h), fix),
            pl.BlockSpec((1, h), fix),
        ]
        args = [hs_h, hd_h, edge_attr, edge_h, Wep, bep2, Wu1, bu12,
                Wu2, bu22, ge2, be2]
        kwargs = {}
        body = _edge_body
        if prev is not None:
            body = lambda p_ref, *refs: _edge_body(*refs)
            in_specs = [pl.BlockSpec(memory_space=pl.ANY)] + in_specs
            args = [prev] + args
            kwargs = dict(input_output_aliases={0: 0})
        return pl.pallas_call(
            body,
            grid=(esz // eb,),
            in_specs=in_specs,
            out_specs=pl.BlockSpec((eb, h), off),
            out_shape=jax.ShapeDtypeStruct((e, h), F32),
            **kwargs,
        )(*args)

    e_new = None
    for soff, esz in slices:
        hs_h, hd_h = _make_gather_kernel(esz, n, h, soff)(x_out, src, dst)
        e_new = edge_part(soff, esz, hs_h, hd_h, e_new)

    return (x_out, e_new)
